# Initial kernel scaffold; baseline (speedup 1.0000x reference)
import jax
import jax.numpy as jnp
from jax.experimental import pallas as pl


def kernel(nodes, edge_index, spin_sites, node_graph_ids, W_enc, W_msg, W_node, W_val, W_pol):
    raise NotImplementedError("write your pallas kernel here")


# speedup vs baseline: 7.2160x; 7.2160x over previous
import functools

import jax
import jax.numpy as jnp
from jax import lax
from jax.experimental import pallas as pl
from jax.experimental.pallas import tpu as pltpu
from jax.experimental.pallas import tpu_sc as plsc

D = 32
_BN = 5000
_W = 80
_SPC = 10
_NS = 16
_NC = 2


def _layer_norm(x, eps=1e-5):
    mu = jnp.mean(x, axis=-1, keepdims=True)
    xc = x - mu
    var = jnp.mean(xc * xc, axis=-1, keepdims=True)
    return xc / jnp.sqrt(var + eps)


def _encode_body(nodes_ref, sites_ref, wenc_ref, wmt_ref, wmb_ref,
                 h_ref, a_ref, b_ref):
    i = pl.program_id(0)
    bn = nodes_ref.shape[0]
    row = lax.broadcasted_iota(jnp.int32, (bn, 128), 0) + i * bn
    is_spin = jnp.max((row == sites_ref[...]).astype(jnp.float32),
                      axis=-1, keepdims=True)
    w = wenc_ref[...]
    h = (nodes_ref[...] * w[0:1, :]
         + (1.0 - is_spin) * w[1:2, :]
         + is_spin * w[2:3, :])
    h = _layer_norm(jnp.maximum(h, 0.0))
    h_ref[...] = h
    a_ref[...] = jnp.dot(h, wmt_ref[...], preferred_element_type=jnp.float32)
    b_ref[...] = jnp.dot(h, wmb_ref[...], preferred_element_type=jnp.float32)


def _update_body(h_ref, agg_ref, wt_ref, wb_ref, wmt_ref, wmb_ref,
                 ho_ref, a_ref, b_ref):
    z = (jnp.dot(h_ref[...], wt_ref[...], preferred_element_type=jnp.float32)
         + jnp.dot(agg_ref[...], wb_ref[...], preferred_element_type=jnp.float32))
    h = _layer_norm(jnp.maximum(z, 0.0))
    ho_ref[...] = h
    a_ref[...] = jnp.dot(h, wmt_ref[...], preferred_element_type=jnp.float32)
    b_ref[...] = jnp.dot(h, wmb_ref[...], preferred_element_type=jnp.float32)


def _final_body(h_ref, agg_ref, wt_ref, wb_ref, ids_ref, sites_ref,
                sum_ref, spin_ref):
    i = pl.program_id(0)
    bn = h_ref.shape[0]
    z = (jnp.dot(h_ref[...], wt_ref[...], preferred_element_type=jnp.float32)
         + jnp.dot(agg_ref[...], wb_ref[...], preferred_element_type=jnp.float32))
    h = _layer_norm(jnp.maximum(z, 0.0))
    gl = lax.broadcasted_iota(jnp.int32, (bn, 128), 1)
    onehot_g = (ids_ref[...] == gl).astype(jnp.float32)
    part_sum = lax.dot_general(onehot_g, h, (((0,), (0,)), ((), ())),
                               preferred_element_type=jnp.float32)
    row = lax.broadcasted_iota(jnp.int32, (bn, 128), 0) + i * bn
    onehot_s = (row == sites_ref[...]).astype(jnp.float32)
    part_spin = lax.dot_general(onehot_s, h, (((0,), (0,)), ((), ())),
                                preferred_element_type=jnp.float32)

    @pl.when(i == 0)
    def _():
        sum_ref[...] = jnp.zeros_like(sum_ref)
        spin_ref[...] = jnp.zeros_like(spin_ref)

    sum_ref[...] += part_sum
    spin_ref[...] += part_spin


def _head_body(g, sum_ref, spin_ref, wv_ref, wp_ref, val_ref, logp_ref):
    ce = jnp.concatenate([sum_ref[...], spin_ref[...]], axis=-1)
    v = jnp.dot(ce, wv_ref[...], preferred_element_type=jnp.float32)
    logits = jnp.dot(ce, wp_ref[...], preferred_element_type=jnp.float32)
    m = jnp.max(logits, axis=-1, keepdims=True)
    lse = m + jnp.log(jnp.sum(jnp.exp(logits - m), axis=-1, keepdims=True))
    lp = logits - lse
    val_ref[...] = v[:g, :]
    logp_ref[...] = lp[:g, :]


def _make_edge_kernel(n, e):
    half = n // 2
    pt = half // _NS
    rows = e // _W
    trows = rows // _NS
    nchunk = trows // _SPC
    mesh = plsc.VectorSubcoreMesh(core_axis_name="c", subcore_axis_name="s",
                                  num_cores=_NC, num_subcores=_NS)

    def body(a_hbm, b_hbm, s_hbm, r_hbm, z_hbm, agg_hbm,
             idx_s, idx_r, idx_l, buf, shared, sem_a, sem_b):
        c = lax.axis_index("c")
        t = lax.axis_index("s")
        base_node = c * half
        pltpu.sync_copy(z_hbm, shared.at[pl.ds(t * pt, pt)])
        plsc.subcore_barrier()
        row_base = t * trows

        def chunk(ci, carry):
            r0 = row_base + ci * _SPC
            pltpu.sync_copy(s_hbm.at[pl.ds(r0, _SPC)], idx_s)
            pltpu.sync_copy(r_hbm.at[pl.ds(r0, _SPC)], idx_r)
            for j in range(_SPC):
                for k in range(_W // 16):
                    r = idx_r[j, k * 16:(k + 1) * 16]
                    loc = r - base_node
                    ok = (loc >= 0) & (loc < half)
                    idx_l[j, k * 16:(k + 1) * 16] = jnp.where(ok, loc, half)
            descs = [pltpu.async_copy(a_hbm.at[idx_s.at[j]], buf.at[j], sem_a)
                     for j in range(_SPC)]
            for d in descs:
                d.wait()
            descs = [pltpu.async_copy(b_hbm.at[idx_r.at[j]], buf.at[j], sem_b,
                                      add=True)
                     for j in range(_SPC)]
            for d in descs:
                d.wait()
            for j in range(_SPC):
                def rbody(i, acc):
                    buf[j, i, 0:16] = jnp.maximum(buf[j, i, 0:16], 0.0)
                    buf[j, i, 16:32] = jnp.maximum(buf[j, i, 16:32], 0.0)
                    return acc
                lax.fori_loop(0, _W, rbody, 0)
            for j in range(_SPC):
                pltpu.sync_copy(buf.at[j], shared.at[idx_l.at[j]], add=True)
            return carry

        lax.fori_loop(0, nchunk, chunk, 0)
        plsc.subcore_barrier()
        pltpu.sync_copy(shared.at[pl.ds(t * pt, pt)],
                        agg_hbm.at[pl.ds(base_node + t * pt, pt)])

    return pl.kernel(
        body,
        out_type=jax.ShapeDtypeStruct((n, D), jnp.float32),
        mesh=mesh,
        compiler_params=pltpu.CompilerParams(use_tc_tiling_on_sc=False),
        scratch_types=[
            pltpu.VMEM((_SPC, _W), jnp.int32),
            pltpu.VMEM((_SPC, _W), jnp.int32),
            pltpu.VMEM((_SPC, _W), jnp.int32),
            pltpu.VMEM((_SPC, _W, D), jnp.float32),
            pltpu.VMEM_SHARED((half + 8, D), jnp.float32),
            pltpu.SemaphoreType.DMA,
            pltpu.SemaphoreType.DMA,
        ],
    )


def kernel(nodes, edge_index, spin_sites, node_graph_ids,
           W_enc, W_msg, W_node, W_val, W_pol):
    n = nodes.shape[0]
    e = edge_index.shape[1]
    g = spin_sites.shape[0]
    nlayers = W_msg.shape[0]
    nblocks = n // _BN

    senders = edge_index[0].reshape(e // _W, _W)
    receivers = edge_index[1].reshape(e // _W, _W)
    sites_pad = jnp.full((1, 128), -1, jnp.int32).at[0, :g].set(spin_sites)
    ids2d = node_graph_ids.reshape(n, 1)
    zeros = jnp.zeros((n // 2 // _NS, D), jnp.float32)

    full = lambda shp: pl.BlockSpec(shp, lambda i: tuple(0 for _ in shp))
    nd_spec = pl.BlockSpec((_BN, D), lambda i: (i, 0))

    encode = pl.pallas_call(
        _encode_body,
        grid=(nblocks,),
        in_specs=[
            pl.BlockSpec((_BN, 1), lambda i: (i, 0)),
            full((1, 128)),
            full((3, D)),
            full((D, D)),
            full((D, D)),
        ],
        out_specs=[nd_spec, nd_spec, nd_spec],
        out_shape=[jax.ShapeDtypeStruct((n, D), jnp.float32)] * 3,
    )

    update = pl.pallas_call(
        _update_body,
        grid=(nblocks,),
        in_specs=[nd_spec, nd_spec, full((D, D)), full((D, D)),
                  full((D, D)), full((D, D))],
        out_specs=[nd_spec, nd_spec, nd_spec],
        out_shape=[jax.ShapeDtypeStruct((n, D), jnp.float32)] * 3,
    )

    final = pl.pallas_call(
        _final_body,
        grid=(nblocks,),
        in_specs=[nd_spec, nd_spec, full((D, D)), full((D, D)),
                  pl.BlockSpec((_BN, 1), lambda i: (i, 0)),
                  full((1, 128))],
        out_specs=[full((128, D)), full((128, D))],
        out_shape=[jax.ShapeDtypeStruct((128, D), jnp.float32)] * 2,
    )

    head = pl.pallas_call(
        functools.partial(_head_body, g),
        out_shape=[jax.ShapeDtypeStruct((g, 1), jnp.float32),
                   jax.ShapeDtypeStruct((g, 2), jnp.float32)],
    )

    edge_agg = _make_edge_kernel(n, e)

    wm_t = [W_msg[l][:D] for l in range(nlayers)]
    wm_b = [W_msg[l][D:] for l in range(nlayers)]
    wn_t = [W_node[l][:D] for l in range(nlayers)]
    wn_b = [W_node[l][D:] for l in range(nlayers)]

    h, a, b = encode(nodes, sites_pad, W_enc, wm_t[0], wm_b[0])
    for l in range(nlayers - 1):
        agg = edge_agg(a, b, senders, receivers, zeros)
        h, a, b = update(h, agg, wn_t[l], wn_b[l], wm_t[l + 1], wm_b[l + 1])
    agg = edge_agg(a, b, senders, receivers, zeros)
    sum_g, spin_g = final(h, agg, wn_t[nlayers - 1], wn_b[nlayers - 1],
                          ids2d, sites_pad)
    values, log_prob = head(sum_g, spin_g, W_val, W_pol)
    return values, log_prob


# speedup vs baseline: 8.4320x; 1.1685x over previous
import functools

import jax
import jax.numpy as jnp
from jax import lax
from jax.experimental import pallas as pl
from jax.experimental.pallas import tpu as pltpu
from jax.experimental.pallas import tpu_sc as plsc

D = 32
_BN = 5000
_W = 80
_SPC = 5
_NS = 16
_NC = 2


def _layer_norm(x, eps=1e-5):
    mu = jnp.mean(x, axis=-1, keepdims=True)
    xc = x - mu
    var = jnp.mean(xc * xc, axis=-1, keepdims=True)
    return xc / jnp.sqrt(var + eps)


def _encode_body(nodes_ref, sites_ref, wenc_ref, wmt_ref, wmb_ref,
                 h_ref, a_ref, b_ref):
    i = pl.program_id(0)
    bn = nodes_ref.shape[0]
    row = lax.broadcasted_iota(jnp.int32, (bn, 128), 0) + i * bn
    is_spin = jnp.max((row == sites_ref[...]).astype(jnp.float32),
                      axis=-1, keepdims=True)
    w = wenc_ref[...]
    h = (nodes_ref[...] * w[0:1, :]
         + (1.0 - is_spin) * w[1:2, :]
         + is_spin * w[2:3, :])
    h = _layer_norm(jnp.maximum(h, 0.0))
    h_ref[...] = h
    a_ref[...] = jnp.dot(h, wmt_ref[...], preferred_element_type=jnp.float32)
    b_ref[...] = jnp.dot(h, wmb_ref[...], preferred_element_type=jnp.float32)


def _update_body(h_ref, agg_ref, wt_ref, wb_ref, wmt_ref, wmb_ref,
                 ho_ref, a_ref, b_ref):
    z = (jnp.dot(h_ref[...], wt_ref[...], preferred_element_type=jnp.float32)
         + jnp.dot(agg_ref[...], wb_ref[...], preferred_element_type=jnp.float32))
    h = _layer_norm(jnp.maximum(z, 0.0))
    ho_ref[...] = h
    a_ref[...] = jnp.dot(h, wmt_ref[...], preferred_element_type=jnp.float32)
    b_ref[...] = jnp.dot(h, wmb_ref[...], preferred_element_type=jnp.float32)


def _final_body(h_ref, agg_ref, wt_ref, wb_ref, ids_ref, sites_ref,
                sum_ref, spin_ref):
    i = pl.program_id(0)
    bn = h_ref.shape[0]
    z = (jnp.dot(h_ref[...], wt_ref[...], preferred_element_type=jnp.float32)
         + jnp.dot(agg_ref[...], wb_ref[...], preferred_element_type=jnp.float32))
    h = _layer_norm(jnp.maximum(z, 0.0))
    gl = lax.broadcasted_iota(jnp.int32, (bn, 128), 1)
    onehot_g = (ids_ref[...] == gl).astype(jnp.float32)
    part_sum = lax.dot_general(onehot_g, h, (((0,), (0,)), ((), ())),
                               preferred_element_type=jnp.float32)
    row = lax.broadcasted_iota(jnp.int32, (bn, 128), 0) + i * bn
    onehot_s = (row == sites_ref[...]).astype(jnp.float32)
    part_spin = lax.dot_general(onehot_s, h, (((0,), (0,)), ((), ())),
                                preferred_element_type=jnp.float32)

    @pl.when(i == 0)
    def _():
        sum_ref[...] = jnp.zeros_like(sum_ref)
        spin_ref[...] = jnp.zeros_like(spin_ref)

    sum_ref[...] += part_sum
    spin_ref[...] += part_spin


def _head_body(g, sum_ref, spin_ref, wv_ref, wp_ref, val_ref, logp_ref):
    ce = jnp.concatenate([sum_ref[...], spin_ref[...]], axis=-1)
    v = jnp.dot(ce, wv_ref[...], preferred_element_type=jnp.float32)
    logits = jnp.dot(ce, wp_ref[...], preferred_element_type=jnp.float32)
    m = jnp.max(logits, axis=-1, keepdims=True)
    lse = m + jnp.log(jnp.sum(jnp.exp(logits - m), axis=-1, keepdims=True))
    lp = logits - lse
    val_ref[...] = v[:g, :]
    logp_ref[...] = lp[:g, :]


def _make_edge_kernel(n, e):
    half = n // 2
    pt = half // _NS
    rows = e // _W
    trows = rows // _NS
    nchunk = trows // _SPC
    mesh = plsc.VectorSubcoreMesh(core_axis_name="c", subcore_axis_name="s",
                                  num_cores=_NC, num_subcores=_NS)


    def body(a_hbm, b_hbm, s_hbm, r_hbm, z_hbm, agg_hbm,
             idx_s, idx_r, idx_l, buf, shared, sem_a, sem_b):
        c = lax.axis_index("c")
        t = lax.axis_index("s")
        base_node = c * half
        pltpu.sync_copy(z_hbm, shared.at[pl.ds(t * pt, pt)])
        plsc.subcore_barrier()
        row_base = t * trows

        def fire_front(p, ci):
            r0 = row_base + ci * _SPC
            pltpu.sync_copy(s_hbm.at[pl.ds(r0, _SPC)], idx_s.at[p])
            pltpu.sync_copy(r_hbm.at[pl.ds(r0, _SPC)], idx_r.at[p])
            for j in range(_SPC):
                for k in range(_W // 16):
                    r = idx_r[p, j, k * 16:(k + 1) * 16]
                    loc = r - base_node
                    ok = (loc >= 0) & (loc < half)
                    idx_l[p, j, k * 16:(k + 1) * 16] = jnp.where(ok, loc, half)
            for j in range(_SPC):
                pltpu.async_copy(a_hbm.at[idx_s.at[p, j]], buf.at[p, j], sem_a)

        def drain_a(p):
            for j in range(_SPC):
                pltpu.make_async_copy(a_hbm.at[idx_s.at[p, j]], buf.at[p, j],
                                      sem_a).wait()

        def finish(p):
            for j in range(_SPC):
                pltpu.make_async_copy(b_hbm.at[idx_r.at[p, j]], buf.at[p, j],
                                      sem_b).wait()
            for j in range(_SPC):
                @plsc.parallel_loop(0, _W, 1, unroll=8)
                def _(i):
                    buf[p, j, i, 0:16] = jnp.maximum(buf[p, j, i, 0:16], 0.0)
                    buf[p, j, i, 16:32] = jnp.maximum(buf[p, j, i, 16:32], 0.0)
            for j in range(_SPC):
                pltpu.sync_copy(buf.at[p, j], shared.at[idx_l.at[p, j]],
                                add=True)

        def fire_b(p):
            for j in range(_SPC):
                pltpu.async_copy(b_hbm.at[idx_r.at[p, j]], buf.at[p, j],
                                 sem_b, add=True)

        def step(p, ci, np_, nci):
            drain_a(p)
            fire_b(p)
            fire_front(np_, nci)
            finish(p)

        fire_front(0, 0)

        def pair(i, carry):
            c0 = 2 * i
            step(0, c0, 1, c0 + 1)
            step(1, c0 + 1, 0, c0 + 2)
            return carry

        pairs = (nchunk - 1) // 2
        lax.fori_loop(0, pairs, pair, 0)
        if nchunk % 2 == 0:
            step(0, 2 * pairs, 1, 2 * pairs + 1)
            drain_a(1)
            fire_b(1)
            finish(1)
        else:
            drain_a(0)
            fire_b(0)
            finish(0)

        plsc.subcore_barrier()
        pltpu.sync_copy(shared.at[pl.ds(t * pt, pt)],
                        agg_hbm.at[pl.ds(base_node + t * pt, pt)])

    return pl.kernel(
        body,
        out_type=jax.ShapeDtypeStruct((n, D), jnp.float32),
        mesh=mesh,
        compiler_params=pltpu.CompilerParams(use_tc_tiling_on_sc=False),
        scratch_types=[
            pltpu.VMEM((2, _SPC, _W), jnp.int32),
            pltpu.VMEM((2, _SPC, _W), jnp.int32),
            pltpu.VMEM((2, _SPC, _W), jnp.int32),
            pltpu.VMEM((2, _SPC, _W, D), jnp.float32),
            pltpu.VMEM_SHARED((half + 8, D), jnp.float32),
            pltpu.SemaphoreType.DMA,
            pltpu.SemaphoreType.DMA,
        ],
    )


def kernel(nodes, edge_index, spin_sites, node_graph_ids,
           W_enc, W_msg, W_node, W_val, W_pol):
    n = nodes.shape[0]
    e = edge_index.shape[1]
    g = spin_sites.shape[0]
    nlayers = W_msg.shape[0]
    nblocks = n // _BN

    senders = edge_index[0].reshape(e // _W, _W)
    receivers = edge_index[1].reshape(e // _W, _W)
    sites_pad = jnp.full((1, 128), -1, jnp.int32).at[0, :g].set(spin_sites)
    ids2d = node_graph_ids.reshape(n, 1)
    zeros = jnp.zeros((n // 2 // _NS, D), jnp.float32)

    full = lambda shp: pl.BlockSpec(shp, lambda i: tuple(0 for _ in shp))
    nd_spec = pl.BlockSpec((_BN, D), lambda i: (i, 0))

    encode = pl.pallas_call(
        _encode_body,
        grid=(nblocks,),
        in_specs=[
            pl.BlockSpec((_BN, 1), lambda i: (i, 0)),
            full((1, 128)),
            full((3, D)),
            full((D, D)),
            full((D, D)),
        ],
        out_specs=[nd_spec, nd_spec, nd_spec],
        out_shape=[jax.ShapeDtypeStruct((n, D), jnp.float32)] * 3,
    )

    update = pl.pallas_call(
        _update_body,
        grid=(nblocks,),
        in_specs=[nd_spec, nd_spec, full((D, D)), full((D, D)),
                  full((D, D)), full((D, D))],
        out_specs=[nd_spec, nd_spec, nd_spec],
        out_shape=[jax.ShapeDtypeStruct((n, D), jnp.float32)] * 3,
    )

    final = pl.pallas_call(
        _final_body,
        grid=(nblocks,),
        in_specs=[nd_spec, nd_spec, full((D, D)), full((D, D)),
                  pl.BlockSpec((_BN, 1), lambda i: (i, 0)),
                  full((1, 128))],
        out_specs=[full((128, D)), full((128, D))],
        out_shape=[jax.ShapeDtypeStruct((128, D), jnp.float32)] * 2,
    )

    head = pl.pallas_call(
        functools.partial(_head_body, g),
        out_shape=[jax.ShapeDtypeStruct((g, 1), jnp.float32),
                   jax.ShapeDtypeStruct((g, 2), jnp.float32)],
    )

    edge_agg = _make_edge_kernel(n, e)

    wm_t = [W_msg[l][:D] for l in range(nlayers)]
    wm_b = [W_msg[l][D:] for l in range(nlayers)]
    wn_t = [W_node[l][:D] for l in range(nlayers)]
    wn_b = [W_node[l][D:] for l in range(nlayers)]

    h, a, b = encode(nodes, sites_pad, W_enc, wm_t[0], wm_b[0])
    for l in range(nlayers - 1):
        agg = edge_agg(a, b, senders, receivers, zeros)
        h, a, b = update(h, agg, wn_t[l], wn_b[l], wm_t[l + 1], wm_b[l + 1])
    agg = edge_agg(a, b, senders, receivers, zeros)
    sum_g, spin_g = final(h, agg, wn_t[nlayers - 1], wn_b[nlayers - 1],
                          ids2d, sites_pad)
    values, log_prob = head(sum_g, spin_g, W_val, W_pol)
    return values, log_prob
